# baseline (speedup 1.0000x reference)
import jax
import jax.numpy as jnp
from jax import lax
from jax.experimental import pallas as pl
from jax.experimental.pallas import tpu as pltpu

BN_EPS = 1e-5
LEAKY_SLOPE = 0.01


def _make_conv_body(H, W, cin, cout):
    hw = H * W

    def _body(x_ref, w_ref, hr_ref, st_ref, p_scr):
        xb = x_ref[0].astype(jnp.bfloat16)
        lane = lax.broadcasted_iota(jnp.int32, (1, hw), 1)
        w_idx = lane % W
        h_idx = lane // W

        for dy in (-1, 0, 1):
            for dx in (-1, 0, 1):
                tap = (dy + 1) * 3 + (dx + 1)
                d = dy * W + dx
                v = xb
                if d != 0:
                    v = pltpu.roll(v, (-d) % hw, axis=1)
                ok = None
                if dy != 0:
                    ok = (h_idx + dy >= 0) & (h_idx + dy < H)
                if dx != 0:
                    c = (w_idx + dx >= 0) & (w_idx + dx < W)
                    ok = c if ok is None else ok & c
                if ok is not None:
                    v = jnp.where(ok, v, jnp.bfloat16(0))
                p_scr[pl.ds(tap * cin, cin), :] = v

        hr = jnp.dot(w_ref[...], p_scr[...],
                     preferred_element_type=jnp.float32)
        hr_ref[0] = hr.astype(jnp.bfloat16)
        h = hr[:cout]
        st_ref[0, 0] = jnp.sum(h, axis=1, keepdims=True)
        st_ref[0, 1] = jnp.sum(h * h, axis=1, keepdims=True)

    return _body


def _make_fin_body(N, H, W, cout, inv_m):
    hw = H * W
    Ho, Wo = H // 2, W // 2

    def _body(hr_ref, st_ref, par_ref, o_ref, u_scr):
        s = st_ref[0, 0]
        s2 = st_ref[0, 1]
        for n in range(1, N):
            s = s + st_ref[n, 0]
            s2 = s2 + st_ref[n, 1]
        mean = s * inv_m
        var = s2 * inv_m - mean * mean
        gs = par_ref[0] * lax.rsqrt(var + BN_EPS)
        scale = 0.25 * gs
        shift = 0.25 * (par_ref[1] - mean * gs)

        hr = hr_ref[0]
        z = hr[:cout].astype(jnp.float32) * scale + shift
        t = jnp.maximum(z, LEAKY_SLOPE * z) + hr[cout:].astype(jnp.float32)
        u = t + pltpu.roll(t, hw - 1, axis=1)
        u = u + pltpu.roll(u, hw - W, axis=1)
        u_scr[...] = jnp.transpose(u)
        parts = [u_scr[pl.ds(2 * W * ho, Wo, 2)] for ho in range(Ho)]
        pooled = jnp.concatenate(parts, axis=0)
        o_ref[0] = jnp.transpose(pooled) + par_ref[2]

    return _body


def kernel(x, w1, b1, gamma, beta, wmix, bmix):
    N, Cin, H, W = x.shape
    Cout = w1.shape[-1]
    Ho, Wo = H // 2, W // 2
    M_in = N * H * W

    x3 = x.reshape(N, Cin, H * W)

    w1rows = jnp.transpose(w1.reshape(9, Cin, Cout), (2, 0, 1))
    w1rows = w1rows.reshape(Cout, 9 * Cin)
    wres = jnp.zeros((Cout, 9 * Cin), jnp.float32)
    wres = lax.dynamic_update_slice(wres, 0.25 * wmix.T, (0, 4 * Cin))
    w_ext = jnp.concatenate([w1rows, wres], axis=0).astype(jnp.bfloat16)

    par = jnp.stack([gamma, beta, bmix], axis=0).astype(jnp.float32)[:, :, None]

    conv_body = _make_conv_body(H, W, Cin, Cout)
    hr_hbm, st_hbm = pl.pallas_call(
        conv_body,
        out_shape=(
            jax.ShapeDtypeStruct((N, 2 * Cout, H * W), jnp.bfloat16),
            jax.ShapeDtypeStruct((N, 2, Cout, 1), jnp.float32),
        ),
        grid=(N,),
        in_specs=[
            pl.BlockSpec((1, Cin, H * W), lambda n: (n, 0, 0)),
            pl.BlockSpec((2 * Cout, 9 * Cin), lambda n: (0, 0)),
        ],
        out_specs=(
            pl.BlockSpec((1, 2 * Cout, H * W), lambda n: (n, 0, 0)),
            pl.BlockSpec((1, 2, Cout, 1), lambda n: (n, 0, 0, 0)),
        ),
        scratch_shapes=[pltpu.VMEM((9 * Cin, H * W), jnp.bfloat16)],
        compiler_params=pltpu.CompilerParams(
            dimension_semantics=("parallel",)),
    )(x3, w_ext)

    fin_body = _make_fin_body(N, H, W, Cout, 1.0 / float(M_in))
    out3 = pl.pallas_call(
        fin_body,
        out_shape=jax.ShapeDtypeStruct((N, Cout, Ho * Wo), jnp.float32),
        grid=(N,),
        in_specs=[
            pl.BlockSpec((1, 2 * Cout, H * W), lambda n: (n, 0, 0)),
            pl.BlockSpec((N, 2, Cout, 1), lambda n: (0, 0, 0, 0)),
            pl.BlockSpec((3, Cout, 1), lambda n: (0, 0, 0)),
        ],
        out_specs=pl.BlockSpec((1, Cout, Ho * Wo), lambda n: (n, 0, 0)),
        scratch_shapes=[pltpu.VMEM((H * W, Cout), jnp.float32)],
        compiler_params=pltpu.CompilerParams(
            dimension_semantics=("parallel",)),
    )(hr_hbm, st_hbm, par)

    return out3.reshape(N, Cout, Ho, Wo)


# speedup vs baseline: 1.0830x; 1.0830x over previous
import jax
import jax.numpy as jnp
from jax import lax
from jax.experimental import pallas as pl
from jax.experimental.pallas import tpu as pltpu

BN_EPS = 1e-5
LEAKY_SLOPE = 0.01


def _make_conv_body(G, H, W, cin, cout):
    hw = H * W

    def _body(x_ref, w_ref, hr_ref, st_ref, p_scr):
        lane = lax.broadcasted_iota(jnp.int32, (1, hw), 1)
        w_idx = lane % W
        h_idx = lane // W

        for g in range(G):
            xb = x_ref[0, g].astype(jnp.bfloat16)
            for dy in (-1, 0, 1):
                for dx in (-1, 0, 1):
                    tap = (dy + 1) * 3 + (dx + 1)
                    d = dy * W + dx
                    v = xb
                    if d != 0:
                        v = pltpu.roll(v, (-d) % hw, axis=1)
                    ok = None
                    if dy != 0:
                        ok = (h_idx + dy >= 0) & (h_idx + dy < H)
                    if dx != 0:
                        c = (w_idx + dx >= 0) & (w_idx + dx < W)
                        ok = c if ok is None else ok & c
                    if ok is not None:
                        v = jnp.where(ok, v, jnp.bfloat16(0))
                    p_scr[pl.ds(tap * cin, cin), pl.ds(g * hw, hw)] = v

        hr = jnp.dot(w_ref[...], p_scr[...],
                     preferred_element_type=jnp.float32)
        hr_ref[0] = hr.astype(jnp.bfloat16)
        h = hr[:cout]
        st_ref[0, 0] = jnp.sum(h, axis=1, keepdims=True)
        st_ref[0, 1] = jnp.sum(h * h, axis=1, keepdims=True)

    return _body


def _make_fin_body(n_tiles, G, H, W, cout, inv_m):
    hw = H * W
    Ho, Wo = H // 2, W // 2

    def _body(hr_ref, st_ref, par_ref, o_ref, u_scr):
        s = st_ref[0, 0]
        s2 = st_ref[0, 1]
        for t in range(1, n_tiles):
            s = s + st_ref[t, 0]
            s2 = s2 + st_ref[t, 1]
        mean = s * inv_m
        var = s2 * inv_m - mean * mean
        gs = par_ref[0] * lax.rsqrt(var + BN_EPS)
        scale = 0.25 * gs
        shift = 0.25 * (par_ref[1] - mean * gs)

        hr = hr_ref[0]
        z = hr[:cout].astype(jnp.float32) * scale + shift
        t_full = (jnp.maximum(z, LEAKY_SLOPE * z)
                  + hr[cout:].astype(jnp.float32))
        u = t_full + pltpu.roll(t_full, G * hw - 1, axis=1)
        u = u + pltpu.roll(u, G * hw - W, axis=1)
        u_scr[...] = jnp.transpose(u)
        for g in range(G):
            parts = [u_scr[pl.ds(g * hw + 2 * W * ho, Wo, 2)]
                     for ho in range(Ho)]
            pooled = jnp.concatenate(parts, axis=0)
            o_ref[0, g] = jnp.transpose(pooled) + par_ref[2]

    return _body


def kernel(x, w1, b1, gamma, beta, wmix, bmix):
    N, Cin, H, W = x.shape
    Cout = w1.shape[-1]
    Ho, Wo = H // 2, W // 2
    M_in = N * H * W

    G = 4 if N % 4 == 0 else 1
    n_tiles = N // G

    x3 = x.reshape(N // G, G, Cin, H * W)

    w1rows = jnp.transpose(w1.reshape(9, Cin, Cout), (2, 0, 1))
    w1rows = w1rows.reshape(Cout, 9 * Cin)
    wres = jnp.zeros((Cout, 9 * Cin), jnp.float32)
    wres = lax.dynamic_update_slice(wres, 0.25 * wmix.T, (0, 4 * Cin))
    w_ext = jnp.concatenate([w1rows, wres], axis=0).astype(jnp.bfloat16)

    par = jnp.stack([gamma, beta, bmix], axis=0).astype(jnp.float32)[:, :, None]

    conv_body = _make_conv_body(G, H, W, Cin, Cout)
    hr_hbm, st_hbm = pl.pallas_call(
        conv_body,
        out_shape=(
            jax.ShapeDtypeStruct((n_tiles, 2 * Cout, G * H * W), jnp.bfloat16),
            jax.ShapeDtypeStruct((n_tiles, 2, Cout, 1), jnp.float32),
        ),
        grid=(n_tiles,),
        in_specs=[
            pl.BlockSpec((1, G, Cin, H * W), lambda t: (t, 0, 0, 0)),
            pl.BlockSpec((2 * Cout, 9 * Cin), lambda t: (0, 0)),
        ],
        out_specs=(
            pl.BlockSpec((1, 2 * Cout, G * H * W), lambda t: (t, 0, 0)),
            pl.BlockSpec((1, 2, Cout, 1), lambda t: (t, 0, 0, 0)),
        ),
        scratch_shapes=[pltpu.VMEM((9 * Cin, G * H * W), jnp.bfloat16)],
        compiler_params=pltpu.CompilerParams(
            dimension_semantics=("parallel",)),
    )(x3, w_ext)

    fin_body = _make_fin_body(n_tiles, G, H, W, Cout, 1.0 / float(M_in))
    out4 = pl.pallas_call(
        fin_body,
        out_shape=jax.ShapeDtypeStruct((n_tiles, G, Cout, Ho * Wo), jnp.float32),
        grid=(n_tiles,),
        in_specs=[
            pl.BlockSpec((1, 2 * Cout, G * H * W), lambda t: (t, 0, 0)),
            pl.BlockSpec((n_tiles, 2, Cout, 1), lambda t: (0, 0, 0, 0)),
            pl.BlockSpec((3, Cout, 1), lambda t: (0, 0, 0)),
        ],
        out_specs=pl.BlockSpec((1, G, Cout, Ho * Wo), lambda t: (t, 0, 0, 0)),
        scratch_shapes=[pltpu.VMEM((G * H * W, Cout), jnp.float32)],
        compiler_params=pltpu.CompilerParams(
            dimension_semantics=("parallel",)),
    )(hr_hbm, st_hbm, par)

    return out4.reshape(N, Cout, Ho, Wo)
